# baseline (speedup 1.0000x reference)
import functools

import jax
import jax.numpy as jnp
from jax import lax
from jax.experimental import pallas as pl
from jax.experimental.pallas import tpu as pltpu
from jax.experimental.pallas import tpu_sc as plsc

_NC = 2
_NS = 16
_NW = _NC * _NS


@functools.lru_cache(maxsize=None)
def _make_gather(B0, H, V, D, S):
    rows_per_w = B0 // _NW
    n_chunks = rows_per_w // S
    C = S * H
    assert n_chunks >= 2 and n_chunks % 2 == 0
    mesh = plsc.VectorSubcoreMesh(
        core_axis_name="c", subcore_axis_name="s",
        num_cores=_NC, num_subcores=_NS,
    )

    @functools.partial(
        pl.kernel,
        out_type=jax.ShapeDtypeStruct((B0, H, D), jnp.float32),
        mesh=mesh,
        scratch_types=[
            [pltpu.VMEM((C,), jnp.int32)] * 2,
            [pltpu.VMEM((C, D), jnp.float32)] * 2,
            [pltpu.SemaphoreType.DMA] * 2,
            [pltpu.SemaphoreType.DMA] * 2,
        ],
        compiler_params=pltpu.CompilerParams(use_tc_tiling_on_sc=False,
                                             needs_layout_passes=False),
    )
    def gather_kernel(x_hbm, table_hbm, out_hbm, idx_v, rows_v, g_sem, st_sem):
        wid = lax.axis_index("s") * _NC + lax.axis_index("c")
        base = wid * rows_per_w

        def load_idx(i, b):
            pltpu.sync_copy(x_hbm.at[pl.ds((base + i * S) * H, C)], idx_v[b])

        def fire_gather(b):
            pltpu.async_copy(table_hbm.at[idx_v[b]], rows_v[b], g_sem[b])

        def wait_gather(b):
            pltpu.make_async_copy(table_hbm.at[idx_v[b]], rows_v[b],
                                  g_sem[b]).wait()

        def fire_store(i, b):
            for s in range(S):
                pltpu.async_copy(rows_v[b].at[pl.ds(s * H, H)],
                                 out_hbm.at[base + i * S + s], st_sem[b])

        def wait_store(b):
            for s in range(S):
                pltpu.make_async_copy(rows_v[b].at[pl.ds(s * H, H)],
                                      out_hbm.at[0], st_sem[b]).wait()

        for b in range(2):
            load_idx(b, b)
            fire_gather(b)

        def body(j, carry):
            for b in range(2):
                i = 2 * j + b
                wait_gather(b)
                fire_store(i - 2, b)
                wait_store(b)
                load_idx(i, b)
                fire_gather(b)
            return carry

        lax.fori_loop(1, n_chunks // 2, body, 0)

        for b in range(2):
            i = n_chunks - 2 + b
            wait_gather(b)
            fire_store(i, b)
        for b in range(2):
            wait_store(b)

    return gather_kernel


def kernel(x, table):
    B0, H = x.shape
    V, D = table.shape
    xf = x.reshape(-1).astype(jnp.int32)
    return _make_gather(B0, H, V, D, 4)(xf, table)


# speedup vs baseline: 1.0407x; 1.0407x over previous
import functools

import jax
import jax.numpy as jnp
from jax import lax
from jax.experimental import pallas as pl
from jax.experimental.pallas import tpu as pltpu
from jax.experimental.pallas import tpu_sc as plsc

_NC = 2
_NS = 16
_NW = _NC * _NS
_L = 16


@functools.lru_cache(maxsize=None)
def _make_gather(B0, H, V2, D):
    rows_per_w = B0 // _NW
    C = H
    n_chunks = rows_per_w
    n_grp = (C + _L - 1) // _L
    mesh = plsc.VectorSubcoreMesh(
        core_axis_name="c", subcore_axis_name="s",
        num_cores=_NC, num_subcores=_NS,
    )

    @functools.partial(
        pl.kernel,
        out_type=jax.ShapeDtypeStruct((B0, H, D), jnp.float32),
        mesh=mesh,
        scratch_types=[
            [pltpu.VMEM((C,), jnp.int32)] * 2,
            [pltpu.VMEM((C,), jnp.int32)] * 2,
            [pltpu.VMEM((C,), jnp.int32)] * 2,
            [pltpu.VMEM((C, 2 * D), jnp.float32)] * 2,
            [pltpu.VMEM((1, H, D), jnp.float32)] * 2,
            [pltpu.SemaphoreType.DMA] * 2,
            [pltpu.SemaphoreType.DMA] * 2,
        ],
        compiler_params=pltpu.CompilerParams(use_tc_tiling_on_sc=True,
                                             needs_layout_passes=False),
    )
    def gather_kernel(x_hbm, t2_hbm, out_hbm, idx_v, idx2_v, par_v, g_v,
                      rows_v, g_sem, st_sem):
        wid = lax.axis_index("s") * _NC + lax.axis_index("c")
        sbase = wid * rows_per_w
        ibase = sbase * H

        def load_prep(i, b):
            pltpu.sync_copy(x_hbm.at[pl.ds(ibase + i * C, C)], idx_v[b])

            def prep(j, carry):
                v = idx_v[b][pl.ds(j * _L, _L)]
                idx2_v[b][pl.ds(j * _L, _L)] = lax.shift_right_logical(v, 1)
                par_v[b][pl.ds(j * _L, _L)] = (v & 1) * D
                return carry

            lax.fori_loop(0, C // _L, prep, 0)
            if C % _L:
                v = idx_v[b][pl.ds(C - _L, _L)]
                idx2_v[b][pl.ds(C - _L, _L)] = lax.shift_right_logical(v, 1)
                par_v[b][pl.ds(C - _L, _L)] = (v & 1) * D

        def fire_gather(b):
            pltpu.async_copy(t2_hbm.at[idx2_v[b]], g_v[b], g_sem[b])

        def wait_gather(b):
            pltpu.make_async_copy(t2_hbm.at[idx2_v[b]], g_v[b],
                                  g_sem[b]).wait()

        def relocate(b):
            zeros = jnp.zeros((_L,), jnp.int32)

            def grp(j, carry):
                rows = j * _L + lax.iota(jnp.int32, _L)
                msk = rows < C
                rows = jnp.where(msk, rows, 0)
                par = plsc.load_gather(par_v[b], [rows])
                for c in range(0, D, _L):
                    cc = c + lax.iota(jnp.int32, _L)
                    vals = plsc.load_gather(g_v[b], [rows, par + c],
                                            mask=msk)
                    plsc.store_scatter(rows_v[b], [zeros, rows, cc],
                                       vals, mask=msk)
                return carry

            lax.fori_loop(0, n_grp, grp, 0)

        def fire_store(i, b):
            pltpu.async_copy(rows_v[b], out_hbm.at[pl.ds(sbase + i, 1)],
                             st_sem[b])

        def wait_store(b):
            pltpu.make_async_copy(rows_v[b], out_hbm.at[pl.ds(0, 1)],
                                  st_sem[b]).wait()

        for b in range(2):
            load_prep(b, b)
            fire_gather(b)

        def body(j, carry):
            for b in range(2):
                i = 2 * j + b
                wait_gather(b)
                relocate(b)
                fire_store(i - 2, b)
                wait_store(b)
                load_prep(i, b)
                fire_gather(b)
            return carry

        lax.fori_loop(1, n_chunks // 2, body, 0)

        for b in range(2):
            i = n_chunks - 2 + b
            wait_gather(b)
            relocate(b)
            fire_store(i, b)
        for b in range(2):
            wait_store(b)

    return gather_kernel


def kernel(x, table):
    B0, H = x.shape
    V, D = table.shape
    xf = x.reshape(-1).astype(jnp.int32)
    t2 = table.reshape(V // 2, 2 * D)
    return _make_gather(B0, H, V // 2, D)(xf, t2)


# speedup vs baseline: 1.2931x; 1.2425x over previous
import functools

import jax
import jax.numpy as jnp
from jax import lax
from jax.experimental import pallas as pl
from jax.experimental.pallas import tpu as pltpu
from jax.experimental.pallas import tpu_sc as plsc

_NC = 2
_NS = 16
_NW = _NC * _NS
_L = 16


@functools.lru_cache(maxsize=None)
def _make_gather(B0, H, V2, D):
    rows_per_w = B0 // _NW
    C = H
    n_chunks = rows_per_w
    n_grp = (C + _L - 1) // _L
    mesh = plsc.VectorSubcoreMesh(
        core_axis_name="c", subcore_axis_name="s",
        num_cores=_NC, num_subcores=_NS,
    )

    @functools.partial(
        pl.kernel,
        out_type=jax.ShapeDtypeStruct((B0, H // 2, 2 * D), jnp.float32),
        mesh=mesh,
        scratch_types=[
            [pltpu.VMEM((C,), jnp.int32)] * 2,
            [pltpu.VMEM((C,), jnp.int32)] * 2,
            [pltpu.VMEM((C,), jnp.int32)] * 2,
            [pltpu.VMEM((C, 2 * D), jnp.float32)] * 2,
            [pltpu.VMEM((1, H // 2, 2 * D), jnp.float32)] * 2,
            [pltpu.SemaphoreType.DMA] * 2,
            [pltpu.SemaphoreType.DMA] * 2,
        ],
        compiler_params=pltpu.CompilerParams(use_tc_tiling_on_sc=True,
                                             needs_layout_passes=False),
    )
    def gather_kernel(x_hbm, t2_hbm, out_hbm, idx_v, idx2_v, par_v, g_v,
                      rows_v, g_sem, st_sem):
        wid = lax.axis_index("s") * _NC + lax.axis_index("c")
        sbase = wid * rows_per_w
        ibase = sbase * H

        def load_prep(i, b):
            pltpu.sync_copy(x_hbm.at[pl.ds(ibase + i * C, C)], idx_v[b])

            def prep(j, carry):
                v = idx_v[b][pl.ds(j * _L, _L)]
                idx2_v[b][pl.ds(j * _L, _L)] = lax.shift_right_logical(v, 1)
                par_v[b][pl.ds(j * _L, _L)] = (v & 1) * D
                return carry

            lax.fori_loop(0, C // _L, prep, 0)
            if C % _L:
                v = idx_v[b][pl.ds(C - _L, _L)]
                idx2_v[b][pl.ds(C - _L, _L)] = lax.shift_right_logical(v, 1)
                par_v[b][pl.ds(C - _L, _L)] = (v & 1) * D

        def fire_gather(b):
            pltpu.async_copy(t2_hbm.at[idx2_v[b]], g_v[b], g_sem[b])

        def wait_gather(b):
            pltpu.make_async_copy(t2_hbm.at[idx2_v[b]], g_v[b],
                                  g_sem[b]).wait()

        def relocate(b):
            zeros = jnp.zeros((_L,), jnp.int32)

            def grp(j, carry):
                rows = j * _L + lax.iota(jnp.int32, _L)
                msk = rows < C
                rows = jnp.where(msk, rows, 0)
                par = plsc.load_gather(par_v[b], [rows])
                half = (rows & 1) * D
                for c in range(0, D, _L):
                    cc = c + lax.iota(jnp.int32, _L)
                    vals = plsc.load_gather(g_v[b], [rows, par + c],
                                            mask=msk)
                    plsc.store_scatter(rows_v[b],
                                       [zeros, lax.shift_right_logical(rows, 1),
                                        half + cc],
                                       vals, mask=msk)
                return carry

            lax.fori_loop(0, n_grp, grp, 0)

        def fire_store(i, b):
            pltpu.async_copy(rows_v[b], out_hbm.at[pl.ds(sbase + i, 1)],
                             st_sem[b])

        def wait_store(b):
            pltpu.make_async_copy(rows_v[b], out_hbm.at[pl.ds(0, 1)],
                                  st_sem[b]).wait()

        for b in range(2):
            load_prep(b, b)
            fire_gather(b)

        def body(j, carry):
            for b in range(2):
                i = 2 * j + b
                wait_gather(b)
                relocate(b)
                fire_store(i - 2, b)
                wait_store(b)
                load_prep(i, b)
                fire_gather(b)
            return carry

        lax.fori_loop(1, n_chunks // 2, body, 0)

        for b in range(2):
            i = n_chunks - 2 + b
            wait_gather(b)
            relocate(b)
            fire_store(i, b)
        for b in range(2):
            wait_store(b)

    return gather_kernel


def kernel(x, table):
    B0, H = x.shape
    V, D = table.shape
    xf = x.reshape(-1).astype(jnp.int32)
    t2 = table.reshape(V // 2, 2 * D)
    out = _make_gather(B0, H, V // 2, D)(xf, t2)
    return out.reshape(B0, H, D)
